# baseline (speedup 1.0000x reference)
import functools

import jax
import jax.numpy as jnp
from jax import lax
from jax.experimental import pallas as pl
from jax.experimental.pallas import tpu as pltpu
from jax.experimental.pallas import tpu_sc as plsc

VOCAB = 1000000
DIM = 64
BSZ = 16384
SEQLEN = 200

NC = 2
NS = 16
NW = NC * NS

BBLK = 512
NQ = BSZ // BBLK
NBLK = SEQLEN * NQ
PER_W = NBLK // NW
NBT = BSZ // 128


def _make_sc_gather():
    mesh = plsc.VectorSubcoreMesh(core_axis_name="c", subcore_axis_name="s")

    @functools.partial(
        pl.kernel,
        mesh=mesh,
        out_type=jax.ShapeDtypeStruct((SEQLEN, BSZ, DIM), jnp.float32),
        scratch_types=[
            pltpu.VMEM((BBLK,), jnp.int32),
            pltpu.VMEM((BBLK,), jnp.int32),
            pltpu.VMEM((BBLK, DIM), jnp.float32),
            pltpu.VMEM((BBLK, DIM), jnp.float32),
            pltpu.SemaphoreType.DMA,
            pltpu.SemaphoreType.DMA,
            pltpu.SemaphoreType.DMA,
            pltpu.SemaphoreType.DMA,
        ],
        compiler_params=pltpu.CompilerParams(use_tc_tiling_on_sc=False),
    )
    def sc_gather(xt_hbm, table_hbm, out_hbm,
                  idx0, idx1, rows0, rows1, g0, g1, o0, o1):
        wid = lax.axis_index("s") * NC + lax.axis_index("c")
        idx_b = (idx0, idx1)
        rows_b = (rows0, rows1)
        gsem = (g0, g1)
        osem = (o0, o1)

        def sq(i):
            g = i * NW + wid
            s = g // NQ
            q = g - s * NQ
            return s, q

        def gstart(i, b):
            s, q = sq(i)
            pltpu.sync_copy(xt_hbm.at[s, pl.ds(q * BBLK, BBLK)], idx_b[b])
            pltpu.async_copy(table_hbm.at[idx_b[b]], rows_b[b], gsem[b])

        def gwait(b):
            pltpu.make_async_copy(
                table_hbm.at[idx_b[b]], rows_b[b], gsem[b]).wait()

        def ostart(i, b):
            s, q = sq(i)
            pltpu.async_copy(
                rows_b[b], out_hbm.at[s, pl.ds(q * BBLK, BBLK), :], osem[b])

        def owait(i, b):
            s, q = sq(i)
            pltpu.make_async_copy(
                rows_b[b], out_hbm.at[s, pl.ds(q * BBLK, BBLK), :],
                osem[b]).wait()

        gstart(0, 0)

        def body(j, carry):
            for b in range(2):
                i = 2 * j + b
                nb = 1 - b
                pl.when(i >= 1)(lambda: owait(i - 1, nb))
                pl.when(i + 1 < PER_W)(lambda: gstart(i + 1, nb))
                gwait(b)
                ostart(i, b)
            return carry

        lax.fori_loop(0, PER_W // 2, body, 0)
        owait(PER_W - 1, 1)

    return sc_gather


def _tc_body(in_ref, out_ref):
    v = in_ref[...]
    rows = v.reshape(128, DIM)
    t = rows.T
    out_ref[...] = t.reshape(8, 8, 128)[None, :, None]


def _make_tc_relayout():
    return pl.pallas_call(
        _tc_body,
        grid=(SEQLEN, NBT),
        in_specs=[pl.BlockSpec((DIM, 128),
                               lambda s, q: (s * NBT + q, 0))],
        out_specs=pl.BlockSpec((1, 8, 1, 8, 128),
                               lambda s, q: (s, 0, q, 0, 0)),
        out_shape=jax.ShapeDtypeStruct((SEQLEN, 8, NBT, 8, 128),
                                       jnp.float32),
    )


_SC_GATHER = _make_sc_gather()
_TC_RELAYOUT = _make_tc_relayout()


def kernel(x, weight):
    inter = _SC_GATHER(x.T, weight)
    flat2 = inter.reshape(SEQLEN * BSZ * DIM // 128, 128)
    out5 = _TC_RELAYOUT(flat2)
    return out5.transpose(2, 4, 0, 1, 3).reshape(BSZ, SEQLEN, DIM)


# speedup vs baseline: 4.1648x; 4.1648x over previous
import functools

import jax
import jax.numpy as jnp
from jax import lax
from jax.experimental import pallas as pl
from jax.experimental.pallas import tpu as pltpu
from jax.experimental.pallas import tpu_sc as plsc

VOCAB = 1000000
DIM = 64
BSZ = 16384
SEQLEN = 200

NC = 2
NS = 16
NW = NC * NS

BBLK = 512
NQ = BSZ // BBLK
NBLK = SEQLEN * NQ
PER_W = NBLK // NW
NBT = BSZ // 128


def _make_sc_gather():
    mesh = plsc.VectorSubcoreMesh(core_axis_name="c", subcore_axis_name="s")

    @functools.partial(
        pl.kernel,
        mesh=mesh,
        out_type=jax.ShapeDtypeStruct((SEQLEN, BSZ, DIM), jnp.float32),
        scratch_types=[
            pltpu.VMEM((BBLK,), jnp.int32),
            pltpu.VMEM((BBLK,), jnp.int32),
            pltpu.VMEM((BBLK, DIM), jnp.float32),
            pltpu.VMEM((BBLK, DIM), jnp.float32),
            pltpu.SemaphoreType.DMA,
            pltpu.SemaphoreType.DMA,
            pltpu.SemaphoreType.DMA,
            pltpu.SemaphoreType.DMA,
        ],
        compiler_params=pltpu.CompilerParams(use_tc_tiling_on_sc=False),
    )
    def sc_gather(xt_hbm, table_hbm, out_hbm,
                  idx0, idx1, rows0, rows1, g0, g1, o0, o1):
        wid = lax.axis_index("s") * NC + lax.axis_index("c")
        idx_b = (idx0, idx1)
        rows_b = (rows0, rows1)
        gsem = (g0, g1)
        osem = (o0, o1)

        def sq(i):
            g = i * NW + wid
            s = g // NQ
            q = g - s * NQ
            return s, q

        def gstart(i, b):
            s, q = sq(i)
            pltpu.sync_copy(xt_hbm.at[s, pl.ds(q * BBLK, BBLK)], idx_b[b])
            pltpu.async_copy(table_hbm.at[idx_b[b]], rows_b[b], gsem[b])

        def gwait(b):
            pltpu.make_async_copy(
                table_hbm.at[idx_b[b]], rows_b[b], gsem[b]).wait()

        def ostart(i, b):
            s, q = sq(i)
            pltpu.async_copy(
                rows_b[b], out_hbm.at[s, pl.ds(q * BBLK, BBLK), :], osem[b])

        def owait(i, b):
            s, q = sq(i)
            pltpu.make_async_copy(
                rows_b[b], out_hbm.at[s, pl.ds(q * BBLK, BBLK), :],
                osem[b]).wait()

        gstart(0, 0)

        def body(j, carry):
            for b in range(2):
                i = 2 * j + b
                nb = 1 - b
                pl.when(i >= 1)(lambda: owait(i - 1, nb))
                pl.when(i + 1 < PER_W)(lambda: gstart(i + 1, nb))
                gwait(b)
                ostart(i, b)
            return carry

        lax.fori_loop(0, PER_W // 2, body, 0)
        owait(PER_W - 1, 1)

    return sc_gather


TPB = 16


def _tc_body(in_ref, out_ref):
    for k in range(TPB):
        v = in_ref[pl.ds(k * DIM, DIM), :]
        rows = v.reshape(128, DIM)
        t = rows.T
        out_ref[0, :, k, :, :] = t.reshape(8, 8, 128)


def _make_tc_relayout():
    return pl.pallas_call(
        _tc_body,
        grid=(SEQLEN, NBT // TPB),
        in_specs=[pl.BlockSpec((TPB * DIM, 128),
                               lambda s, q: (s * (NBT // TPB) + q, 0))],
        out_specs=pl.BlockSpec((1, 8, TPB, 8, 128),
                               lambda s, q: (s, 0, q, 0, 0)),
        out_shape=jax.ShapeDtypeStruct((SEQLEN, 8, NBT, 8, 128),
                                       jnp.float32),
        compiler_params=pltpu.CompilerParams(
            vmem_limit_bytes=100 * 1024 * 1024),
    )


_SC_GATHER = _make_sc_gather()
_TC_RELAYOUT = _make_tc_relayout()


def kernel(x, weight):
    inter = _SC_GATHER(x.T, weight)
    flat2 = inter.reshape(SEQLEN * BSZ * DIM // 128, 128)
    out5 = _TC_RELAYOUT(flat2)
    return out5.transpose(2, 4, 0, 1, 3).reshape(BSZ, SEQLEN, DIM)


# speedup vs baseline: 4.8805x; 1.1718x over previous
import functools

import jax
import jax.numpy as jnp
from jax import lax
from jax.experimental import pallas as pl
from jax.experimental.pallas import tpu as pltpu
from jax.experimental.pallas import tpu_sc as plsc

VOCAB = 1000000
DIM = 64
BSZ = 16384
SEQLEN = 200

NC = 2
NS = 16
NW = NC * NS

BBLK = 512
NQ = BSZ // BBLK
NBLK = SEQLEN * NQ
PER_W = NBLK // NW
NBT = BSZ // 128


def _make_sc_gather():
    mesh = plsc.VectorSubcoreMesh(core_axis_name="c", subcore_axis_name="s")

    @functools.partial(
        pl.kernel,
        mesh=mesh,
        out_type=jax.ShapeDtypeStruct((SEQLEN, BSZ, DIM), jnp.float32),
        scratch_types=[
            pltpu.VMEM((BBLK,), jnp.int32),
            pltpu.VMEM((BBLK,), jnp.int32),
            pltpu.VMEM((BBLK, DIM), jnp.float32),
            pltpu.VMEM((BBLK, DIM), jnp.float32),
            pltpu.SemaphoreType.DMA,
            pltpu.SemaphoreType.DMA,
            pltpu.SemaphoreType.DMA,
            pltpu.SemaphoreType.DMA,
        ],
        compiler_params=pltpu.CompilerParams(use_tc_tiling_on_sc=False),
    )
    def sc_gather(xt_hbm, table_hbm, out_hbm,
                  idx0, idx1, rows0, rows1, g0, g1, o0, o1):
        wid = lax.axis_index("s") * NC + lax.axis_index("c")
        idx_b = (idx0, idx1)
        rows_b = (rows0, rows1)
        gsem = (g0, g1)
        osem = (o0, o1)

        def sq(i):
            g = i * NW + wid
            s = g // NQ
            q = g - s * NQ
            return s, q

        def gstart(i, b):
            s, q = sq(i)
            pltpu.sync_copy(xt_hbm.at[s, pl.ds(q * BBLK, BBLK)], idx_b[b])
            pltpu.async_copy(table_hbm.at[idx_b[b]], rows_b[b], gsem[b])

        def gwait(b):
            pltpu.make_async_copy(
                table_hbm.at[idx_b[b]], rows_b[b], gsem[b]).wait()

        def ostart(i, b):
            s, q = sq(i)
            pltpu.async_copy(
                rows_b[b], out_hbm.at[s, pl.ds(q * BBLK, BBLK), :], osem[b])

        def owait(i, b):
            s, q = sq(i)
            pltpu.make_async_copy(
                rows_b[b], out_hbm.at[s, pl.ds(q * BBLK, BBLK), :],
                osem[b]).wait()

        gstart(0, 0)

        def body(j, carry):
            for b in range(2):
                i = 2 * j + b
                nb = 1 - b
                pl.when(i >= 1)(lambda: owait(i - 1, nb))
                pl.when(i + 1 < PER_W)(lambda: gstart(i + 1, nb))
                gwait(b)
                ostart(i, b)
            return carry

        lax.fori_loop(0, PER_W // 2, body, 0)
        owait(PER_W - 1, 1)

    return sc_gather


TPB = 16


def _tc_body(in_ref, out_ref):
    ident = jax.lax.broadcasted_iota(jnp.int32, (128, 128), 0)
    ident = (ident == jax.lax.broadcasted_iota(jnp.int32, (128, 128), 1))
    ident = ident.astype(jnp.float32)
    for k in range(TPB):
        v = in_ref[pl.ds(k * DIM, DIM), :]
        rows = v.reshape(128, DIM)
        t = jax.lax.dot_general(
            rows, ident, (((0,), (0,)), ((), ())),
            preferred_element_type=jnp.float32)
        out_ref[0, :, k, :, :] = t.reshape(8, 8, 128)


def _make_tc_relayout():
    return pl.pallas_call(
        _tc_body,
        grid=(SEQLEN, NBT // TPB),
        in_specs=[pl.BlockSpec((TPB * DIM, 128),
                               lambda s, q: (s * (NBT // TPB) + q, 0))],
        out_specs=pl.BlockSpec((1, 8, TPB, 8, 128),
                               lambda s, q: (s, 0, q, 0, 0)),
        out_shape=jax.ShapeDtypeStruct((SEQLEN, 8, NBT, 8, 128),
                                       jnp.float32),
        compiler_params=pltpu.CompilerParams(
            vmem_limit_bytes=100 * 1024 * 1024),
    )


_SC_GATHER = _make_sc_gather()
_TC_RELAYOUT = _make_tc_relayout()


def kernel(x, weight):
    inter = _SC_GATHER(x.T, weight)
    flat2 = inter.reshape(SEQLEN * BSZ * DIM // 128, 128)
    out5 = _TC_RELAYOUT(flat2)
    return out5.transpose(2, 4, 0, 1, 3).reshape(BSZ, SEQLEN, DIM)


# speedup vs baseline: 5.8244x; 1.1934x over previous
import functools

import jax
import jax.numpy as jnp
from jax import lax
from jax.experimental import pallas as pl
from jax.experimental.pallas import tpu as pltpu
from jax.experimental.pallas import tpu_sc as plsc

VOCAB = 1000000
DIM = 64
BSZ = 16384
SEQLEN = 200

NC = 2
NS = 16
NW = NC * NS

BBLK = 512
NQ = BSZ // BBLK
NBLK = SEQLEN * NQ
PER_W = NBLK // NW
NBT = BSZ // 128


def _make_sc_gather():
    mesh = plsc.VectorSubcoreMesh(core_axis_name="c", subcore_axis_name="s")

    @functools.partial(
        pl.kernel,
        mesh=mesh,
        out_type=jax.ShapeDtypeStruct((SEQLEN, BSZ, DIM), jnp.float32),
        scratch_types=[
            pltpu.VMEM((BBLK,), jnp.int32),
            pltpu.VMEM((BBLK,), jnp.int32),
            pltpu.VMEM((BBLK, DIM), jnp.float32),
            pltpu.VMEM((BBLK, DIM), jnp.float32),
            pltpu.SemaphoreType.DMA,
            pltpu.SemaphoreType.DMA,
            pltpu.SemaphoreType.DMA,
            pltpu.SemaphoreType.DMA,
        ],
        compiler_params=pltpu.CompilerParams(use_tc_tiling_on_sc=False),
    )
    def sc_gather(xt_hbm, table_hbm, out_hbm,
                  idx0, idx1, rows0, rows1, g0, g1, o0, o1):
        wid = lax.axis_index("s") * NC + lax.axis_index("c")
        idx_b = (idx0, idx1)
        rows_b = (rows0, rows1)
        gsem = (g0, g1)
        osem = (o0, o1)

        def sq(i):
            g = i * NW + wid
            s = g // NQ
            q = g - s * NQ
            return s, q

        def gstart(i, b):
            s, q = sq(i)
            pltpu.sync_copy(xt_hbm.at[s, pl.ds(q * BBLK, BBLK)], idx_b[b])
            pltpu.async_copy(table_hbm.at[idx_b[b]], rows_b[b], gsem[b])

        def gwait(b):
            pltpu.make_async_copy(
                table_hbm.at[idx_b[b]], rows_b[b], gsem[b]).wait()

        def ostart(i, b):
            s, q = sq(i)
            pltpu.async_copy(
                rows_b[b], out_hbm.at[s, pl.ds(q * BBLK, BBLK), :], osem[b])

        def owait(i, b):
            s, q = sq(i)
            pltpu.make_async_copy(
                rows_b[b], out_hbm.at[s, pl.ds(q * BBLK, BBLK), :],
                osem[b]).wait()

        gstart(0, 0)

        def body(j, carry):
            for b in range(2):
                i = 2 * j + b
                nb = 1 - b
                pl.when(i >= 1)(lambda: owait(i - 1, nb))
                pl.when(i + 1 < PER_W)(lambda: gstart(i + 1, nb))
                gwait(b)
                ostart(i, b)
            return carry

        lax.fori_loop(0, PER_W // 2, body, 0)
        owait(PER_W - 1, 1)

    return sc_gather


TPB = 64


def _tc_body(in_ref, out_ref):
    ident = jax.lax.broadcasted_iota(jnp.int32, (128, 128), 0)
    ident = (ident == jax.lax.broadcasted_iota(jnp.int32, (128, 128), 1))
    ident = ident.astype(jnp.float32)
    for k in range(TPB):
        v = in_ref[pl.ds(k * DIM, DIM), :]
        rows = v.reshape(128, DIM)
        t = jax.lax.dot_general(
            rows, ident, (((0,), (0,)), ((), ())),
            preferred_element_type=jnp.float32)
        out_ref[0, :, k, :, :] = t.reshape(8, 8, 128)


def _make_tc_relayout():
    return pl.pallas_call(
        _tc_body,
        grid=(SEQLEN, NBT // TPB),
        in_specs=[pl.BlockSpec((TPB * DIM, 128),
                               lambda s, q: (s * (NBT // TPB) + q, 0))],
        out_specs=pl.BlockSpec((1, 8, TPB, 8, 128),
                               lambda s, q: (s, 0, q, 0, 0)),
        out_shape=jax.ShapeDtypeStruct((SEQLEN, 8, NBT, 8, 128),
                                       jnp.float32),
        compiler_params=pltpu.CompilerParams(
            vmem_limit_bytes=100 * 1024 * 1024),
    )


_SC_GATHER = _make_sc_gather()
_TC_RELAYOUT = _make_tc_relayout()


def kernel(x, weight):
    inter = _SC_GATHER(x.T, weight)
    flat2 = inter.reshape(SEQLEN * BSZ * DIM // 128, 128)
    out5 = _TC_RELAYOUT(flat2)
    return out5.transpose(2, 4, 0, 1, 3).reshape(BSZ, SEQLEN, DIM)


# speedup vs baseline: 5.9317x; 1.0184x over previous
import functools

import jax
import jax.numpy as jnp
from jax import lax
from jax.experimental import pallas as pl
from jax.experimental.pallas import tpu as pltpu
from jax.experimental.pallas import tpu_sc as plsc

VOCAB = 1000000
DIM = 64
BSZ = 16384
SEQLEN = 200

NC = 2
NS = 16
NW = NC * NS

BBLK = 512
NQ = BSZ // BBLK
NBLK = SEQLEN * NQ
PER_W = NBLK // NW
NBT = BSZ // 128


def _make_sc_gather():
    mesh = plsc.VectorSubcoreMesh(core_axis_name="c", subcore_axis_name="s")

    @functools.partial(
        pl.kernel,
        mesh=mesh,
        out_type=jax.ShapeDtypeStruct((SEQLEN, BSZ, DIM), jnp.float32),
        scratch_types=[
            pltpu.VMEM((BBLK,), jnp.int32),
            pltpu.VMEM((BBLK,), jnp.int32),
            pltpu.VMEM((BBLK, DIM), jnp.float32),
            pltpu.VMEM((BBLK, DIM), jnp.float32),
            pltpu.SemaphoreType.DMA,
            pltpu.SemaphoreType.DMA,
            pltpu.SemaphoreType.DMA,
            pltpu.SemaphoreType.DMA,
        ],
        compiler_params=pltpu.CompilerParams(use_tc_tiling_on_sc=False),
    )
    def sc_gather(xt_hbm, table_hbm, out_hbm,
                  idx0, idx1, rows0, rows1, g0, g1, o0, o1):
        wid = lax.axis_index("s") * NC + lax.axis_index("c")
        idx_b = (idx0, idx1)
        rows_b = (rows0, rows1)
        gsem = (g0, g1)
        osem = (o0, o1)

        def sq(i):
            g = i * NW + wid
            s = g // NQ
            q = g - s * NQ
            return s, q

        def gstart(i, b):
            s, q = sq(i)
            pltpu.sync_copy(xt_hbm.at[s, pl.ds(q * BBLK, BBLK)], idx_b[b])
            pltpu.async_copy(table_hbm.at[idx_b[b]], rows_b[b], gsem[b])

        def gwait(b):
            pltpu.make_async_copy(
                table_hbm.at[idx_b[b]], rows_b[b], gsem[b]).wait()

        def ostart(i, b):
            s, q = sq(i)
            pltpu.async_copy(
                rows_b[b], out_hbm.at[s, pl.ds(q * BBLK, BBLK), :], osem[b])

        def owait(i, b):
            s, q = sq(i)
            pltpu.make_async_copy(
                rows_b[b], out_hbm.at[s, pl.ds(q * BBLK, BBLK), :],
                osem[b]).wait()

        gstart(0, 0)

        def body(j, carry):
            for b in range(2):
                i = 2 * j + b
                nb = 1 - b
                pl.when(i >= 1)(lambda: owait(i - 1, nb))
                pl.when(i + 1 < PER_W)(lambda: gstart(i + 1, nb))
                gwait(b)
                ostart(i, b)
            return carry

        lax.fori_loop(0, PER_W // 2, body, 0)
        owait(PER_W - 1, 1)

    return sc_gather


TPB = 128


def _tc_body(in_ref, out_ref):
    ident = jax.lax.broadcasted_iota(jnp.int32, (128, 128), 0)
    ident = (ident == jax.lax.broadcasted_iota(jnp.int32, (128, 128), 1))
    ident = ident.astype(jnp.float32)
    for k in range(TPB):
        v = in_ref[pl.ds(k * DIM, DIM), :]
        rows = v.reshape(128, DIM)
        t = jax.lax.dot_general(
            rows, ident, (((0,), (0,)), ((), ())),
            preferred_element_type=jnp.float32)
        out_ref[0, :, k, :, :] = t.reshape(8, 8, 128)


def _make_tc_relayout():
    return pl.pallas_call(
        _tc_body,
        grid=(SEQLEN, NBT // TPB),
        in_specs=[pl.BlockSpec((TPB * DIM, 128),
                               lambda s, q: (s * (NBT // TPB) + q, 0))],
        out_specs=pl.BlockSpec((1, 8, TPB, 8, 128),
                               lambda s, q: (s, 0, q, 0, 0)),
        out_shape=jax.ShapeDtypeStruct((SEQLEN, 8, NBT, 8, 128),
                                       jnp.float32),
        compiler_params=pltpu.CompilerParams(
            vmem_limit_bytes=100 * 1024 * 1024),
    )


_SC_GATHER = _make_sc_gather()
_TC_RELAYOUT = _make_tc_relayout()


def kernel(x, weight):
    inter = _SC_GATHER(x.T, weight)
    flat2 = inter.reshape(SEQLEN * BSZ * DIM // 128, 128)
    out5 = _TC_RELAYOUT(flat2)
    return out5.transpose(2, 4, 0, 1, 3).reshape(BSZ, SEQLEN, DIM)
